# Initial kernel scaffold; baseline (speedup 1.0000x reference)
import jax
import jax.numpy as jnp
from jax.experimental import pallas as pl


def kernel(tx_x, entity_x, edge_index, emb_tables, tc_Wq, tc_bq, tc_Wk, tc_bk, tc_Wv, tc_bv, tc_Ws, tc_bs, ln_g, ln_b, cls_W1, cls_b1, cls_W2, cls_b2, cls_W3, cls_b3):
    raise NotImplementedError("write your pallas kernel here")


# speedup vs baseline: 2.0285x; 2.0285x over previous
import functools

import jax
import jax.numpy as jnp
from jax import lax
from jax.experimental import pallas as pl
from jax.experimental.pallas import tpu as pltpu
from jax.experimental.pallas import tpu_sc as plsc

T = 11
N = 20000
NT = T * N
E = 50000
D = 128
F = 394
NTX = 50000

NC = 2
NS = 16
L = 16
NW = NC * NS

S1_CHUNK = 128
S1_PER_W = 6912
NT_PAD = NW * S1_PER_W

EP = 51200
ETP = T * EP
S2_PER_W = ETP // NW
S2_CHUNK = 160
S3_PER_TILE = EP // NS
S3_CHUNK = 160
S4_PER_TILE = ETP // NS
S4_CHUNK = 160
ROWS_PER_TILE = N // NS
ZROWS = 125
IGNORED = N

_MESH = plsc.VectorSubcoreMesh(core_axis_name="c", subcore_axis_name="s")


def _s1_body(table, idx_hbm, out_hbm, idxb, rows, sem):
    c = lax.axis_index("c")
    s = lax.axis_index("s")
    wid = s * NC + c

    def chunk(i, carry):
        base = wid * S1_PER_W + i * S1_CHUNK
        pltpu.sync_copy(idx_hbm.at[pl.ds(base, S1_CHUNK)], idxb)
        pltpu.async_copy(table.at[idxb], rows, sem).wait()
        pltpu.sync_copy(rows, out_hbm.at[pl.ds(base, S1_CHUNK), :])
        return carry

    lax.fori_loop(0, S1_PER_W // S1_CHUNK, chunk, 0)


_s1 = pl.kernel(
    _s1_body,
    out_type=jax.ShapeDtypeStruct((NT_PAD, D), jnp.float32),
    mesh=_MESH,
    compiler_params=pltpu.CompilerParams(
        needs_layout_passes=False, use_tc_tiling_on_sc=False),
    scratch_types=[
        pltpu.VMEM((S1_CHUNK,), jnp.int32),
        pltpu.VMEM((S1_CHUNK, D), jnp.float32),
        pltpu.SemaphoreType.DMA,
    ],
)


def _s2_body(q_hbm, k_hbm, qidx_hbm, kidx_hbm, ex_hbm, qib, kib, qb, kb, exb, sem):
    c = lax.axis_index("c")
    s = lax.axis_index("s")
    wid = s * NC + c
    inv_sqrt_d = 1.0 / (float(D) ** 0.5)

    def chunk(i, carry):
        base = wid * S2_PER_W + i * S2_CHUNK
        pltpu.sync_copy(qidx_hbm.at[pl.ds(base, S2_CHUNK)], qib)
        pltpu.sync_copy(kidx_hbm.at[pl.ds(base, S2_CHUNK)], kib)
        pltpu.async_copy(q_hbm.at[qib], qb, sem).wait()
        pltpu.async_copy(k_hbm.at[kib], kb, sem).wait()
        for g in range(S2_CHUNK // L):
            rows = lax.iota(jnp.int32, L) + g * L

            def dot_step(d, acc):
                cols = jnp.zeros((L,), jnp.int32) + d
                qv = plsc.load_gather(qb, [rows, cols])
                kv = plsc.load_gather(kb, [rows, cols])
                return acc + qv * kv

            acc = lax.fori_loop(0, D, dot_step, jnp.zeros((L,), jnp.float32),
                                unroll=8)
            exv = jnp.exp(acc * inv_sqrt_d)
            plsc.store_scatter(exb, [rows, jnp.zeros((L,), jnp.int32)], exv)
        pltpu.sync_copy(exb, ex_hbm.at[pl.ds(base, S2_CHUNK), :])
        return carry

    lax.fori_loop(0, S2_PER_W // S2_CHUNK, chunk, 0)


_s2 = pl.kernel(
    _s2_body,
    out_type=jax.ShapeDtypeStruct((ETP, 1), jnp.float32),
    mesh=_MESH,
    compiler_params=pltpu.CompilerParams(
        needs_layout_passes=False, use_tc_tiling_on_sc=False),
    scratch_types=[
        pltpu.VMEM((S2_CHUNK,), jnp.int32),
        pltpu.VMEM((S2_CHUNK,), jnp.int32),
        pltpu.VMEM((S2_CHUNK, D), jnp.float32),
        pltpu.VMEM((S2_CHUNK, D), jnp.float32),
        pltpu.VMEM((S2_CHUNK, 1), jnp.float32),
        pltpu.SemaphoreType.DMA,
    ],
)


def _s3_body(v2, esrc, edst, ex2, zrow, zden, numer, den,
             srcb, dstb, exb, exb16, vb, zb, accum, dacc, sem):
    c = lax.axis_index("c")
    s = lax.axis_index("s")
    pltpu.sync_copy(zrow, zb)
    pltpu.sync_copy(zden.at[pl.ds(0, S3_CHUNK), :], exb16)
    row0 = s * ROWS_PER_TILE

    def per_type(t, carry):
        plsc.subcore_barrier()
        for j in range(ROWS_PER_TILE // ZROWS):
            pltpu.sync_copy(zb, accum.at[pl.ds(row0 + j * ZROWS, ZROWS), :])

        @pl.when(c == 0)
        def _():
            pltpu.sync_copy(zden, dacc.at[pl.ds(row0, ROWS_PER_TILE), :])

        plsc.subcore_barrier()

        def chunk(i, carry2):
            base = t * EP + s * S3_PER_TILE + i * S3_CHUNK
            pltpu.sync_copy(esrc.at[pl.ds(base, S3_CHUNK)], srcb)
            pltpu.sync_copy(edst.at[pl.ds(base, S3_CHUNK)], dstb)
            pltpu.sync_copy(ex2.at[pl.ds(base, S3_CHUNK), :], exb)
            pltpu.async_copy(v2.at[c].at[srcb], vb, sem).wait()
            for g in range(S3_CHUNK // L):
                rows = lax.iota(jnp.int32, L) + g * L
                zc = jnp.zeros((L,), jnp.int32)
                exv = plsc.load_gather(exb, [rows, zc])
                plsc.store_scatter(exb16, [rows, zc], exv)
                for col in range(64):
                    cols = jnp.full((L,), col, jnp.int32)
                    vv = plsc.load_gather(vb, [rows, cols])
                    plsc.store_scatter(vb, [rows, cols], vv * exv)
            pltpu.sync_copy(
                vb, accum.at[plsc.Indices(dstb, ignored_value=IGNORED)],
                add=True)

            @pl.when(c == 0)
            def _():
                pltpu.sync_copy(
                    exb16, dacc.at[plsc.Indices(dstb, ignored_value=IGNORED)],
                    add=True)

            return carry2

        lax.fori_loop(0, S3_PER_TILE // S3_CHUNK, chunk, 0)
        plsc.subcore_barrier()
        pltpu.sync_copy(accum.at[pl.ds(row0, ROWS_PER_TILE), :],
                        numer.at[c, t, pl.ds(row0, ROWS_PER_TILE), :])

        @pl.when(c == 0)
        def _():
            pltpu.sync_copy(dacc.at[pl.ds(row0, ROWS_PER_TILE), :],
                            den.at[t, pl.ds(row0, ROWS_PER_TILE), :])

        return carry

    lax.fori_loop(0, T, per_type, 0)


_s3 = pl.kernel(
    _s3_body,
    out_type=(
        jax.ShapeDtypeStruct((NC, T, N, 64), jnp.float32),
        jax.ShapeDtypeStruct((T, N, 16), jnp.float32),
    ),
    mesh=_MESH,
    compiler_params=pltpu.CompilerParams(
        needs_layout_passes=False, use_tc_tiling_on_sc=False),
    scratch_types=[
        pltpu.VMEM((S3_CHUNK,), jnp.int32),
        pltpu.VMEM((S3_CHUNK,), jnp.int32),
        pltpu.VMEM((S3_CHUNK, 1), jnp.float32),
        pltpu.VMEM((S3_CHUNK, 16), jnp.float32),
        pltpu.VMEM((S3_CHUNK, 64), jnp.float32),
        pltpu.VMEM((ZROWS, 64), jnp.float32),
        pltpu.VMEM_SHARED((N, 64), jnp.float32),
        pltpu.VMEM_SHARED((N, 16), jnp.float32),
        pltpu.SemaphoreType.DMA,
    ],
)


def _s4_body(p2, esrc, edst, zrow, agg, srcb, dstb, pb, zb, accum, sem):
    c = lax.axis_index("c")
    s = lax.axis_index("s")
    pltpu.sync_copy(zrow, zb)
    row0 = s * ROWS_PER_TILE
    for j in range(ROWS_PER_TILE // ZROWS):
        pltpu.sync_copy(zb, accum.at[pl.ds(row0 + j * ZROWS, ZROWS), :])
    plsc.subcore_barrier()

    def chunk(i, carry):
        base = s * S4_PER_TILE + i * S4_CHUNK
        pltpu.sync_copy(esrc.at[pl.ds(base, S4_CHUNK)], srcb)
        pltpu.sync_copy(edst.at[pl.ds(base, S4_CHUNK)], dstb)
        pltpu.async_copy(p2.at[c].at[srcb], pb, sem).wait()
        pltpu.sync_copy(
            pb, accum.at[plsc.Indices(dstb, ignored_value=IGNORED)], add=True)
        return carry

    lax.fori_loop(0, S4_PER_TILE // S4_CHUNK, chunk, 0)
    plsc.subcore_barrier()
    pltpu.sync_copy(accum.at[pl.ds(row0, ROWS_PER_TILE), :],
                    agg.at[c, pl.ds(row0, ROWS_PER_TILE), :])


_s4 = pl.kernel(
    _s4_body,
    out_type=jax.ShapeDtypeStruct((NC, N, 64), jnp.float32),
    mesh=_MESH,
    compiler_params=pltpu.CompilerParams(
        needs_layout_passes=False, use_tc_tiling_on_sc=False),
    scratch_types=[
        pltpu.VMEM((S4_CHUNK,), jnp.int32),
        pltpu.VMEM((S4_CHUNK,), jnp.int32),
        pltpu.VMEM((S4_CHUNK, 64), jnp.float32),
        pltpu.VMEM((ZROWS, 64), jnp.float32),
        pltpu.VMEM_SHARED((N, 64), jnp.float32),
        pltpu.SemaphoreType.DMA,
    ],
)


_T1B = 2048


def _t1_body(x_ref, w_ref, b_ref, q_ref, k_ref, v2_ref, s_ref):
    y = jnp.dot(x_ref[...], w_ref[...],
                preferred_element_type=jnp.float32) + b_ref[...]
    q_ref[...] = y[:, :128]
    k_ref[...] = y[:, 128:256]
    v2_ref[0] = y[:, 256:320]
    v2_ref[1] = y[:, 320:384]
    s_ref[...] = y[:, 384:]


def _t1(h0, wt, bcat):
    grid = (NT_PAD // _T1B,)
    return pl.pallas_call(
        _t1_body,
        grid=grid,
        in_specs=[
            pl.BlockSpec((_T1B, D), lambda i: (i, 0)),
            pl.BlockSpec((D, 512), lambda i: (0, 0)),
            pl.BlockSpec((1, 512), lambda i: (0, 0)),
        ],
        out_specs=[
            pl.BlockSpec((_T1B, D), lambda i: (i, 0)),
            pl.BlockSpec((_T1B, D), lambda i: (i, 0)),
            pl.BlockSpec((NC, _T1B, 64), lambda i: (0, i, 0)),
            pl.BlockSpec((_T1B, D), lambda i: (i, 0)),
        ],
        out_shape=[
            jax.ShapeDtypeStruct((NT_PAD, D), jnp.float32),
            jax.ShapeDtypeStruct((NT_PAD, D), jnp.float32),
            jax.ShapeDtypeStruct((NC, NT_PAD, 64), jnp.float32),
            jax.ShapeDtypeStruct((NT_PAD, D), jnp.float32),
        ],
    )(h0, wt, bcat)


_T2B = 2000


def _t2_body(n_ref, d_ref, s_ref, g_ref, b_ref, w_ref, p_ref):
    nl = n_ref[0, 0]
    nh = n_ref[1, 0]
    den = d_ref[0, :, :1] + 1e-16
    h = jnp.concatenate([nl, nh], axis=-1) / den + s_ref[0]
    mu = jnp.mean(h, axis=-1, keepdims=True)
    r = h - mu
    var = jnp.mean(r * r, axis=-1, keepdims=True)
    hn = r / jnp.sqrt(var + 1e-5) * g_ref[0, 0] + b_ref[0, 0]
    p = jnp.dot(hn, w_ref[0], preferred_element_type=jnp.float32)
    p_ref[0, 0] = p[:, :64]
    p_ref[1, 0] = p[:, 64:]


def _t2(numer, den, sres, g, b, w1t):
    grid = (T, N // _T2B)
    return pl.pallas_call(
        _t2_body,
        grid=grid,
        in_specs=[
            pl.BlockSpec((NC, 1, _T2B, 64), lambda t, r: (0, t, r, 0)),
            pl.BlockSpec((1, _T2B, 16), lambda t, r: (t, r, 0)),
            pl.BlockSpec((1, _T2B, D), lambda t, r: (t, r, 0)),
            pl.BlockSpec((1, 1, D), lambda t, r: (t, 0, 0)),
            pl.BlockSpec((1, 1, D), lambda t, r: (t, 0, 0)),
            pl.BlockSpec((1, D, D), lambda t, r: (t, 0, 0)),
        ],
        out_specs=pl.BlockSpec((NC, 1, _T2B, 64), lambda t, r: (0, t, r, 0)),
        out_shape=jax.ShapeDtypeStruct((NC, T, N, 64), jnp.float32),
    )(numer, den, sres, g, b, w1t)


_T3B = 2000


def _t3_body(x_ref, w1_ref, b1_ref, a_ref, w2_ref, b2_ref, w3_ref, b3_ref,
             o_ref):
    i = pl.program_id(0)
    acc = jnp.dot(x_ref[...], w1_ref[...],
                  preferred_element_type=jnp.float32) + b1_ref[...]
    acc = acc + jnp.where(i < N // _T3B, 1.0, 0.0) * a_ref[...]
    z = jnp.maximum(acc, 0.0)
    z2 = jnp.maximum(
        jnp.dot(z, w2_ref[...], preferred_element_type=jnp.float32)
        + b2_ref[...], 0.0)
    o_ref[...] = jnp.sum(z2 * w3_ref[...], axis=-1, keepdims=True) + b3_ref[0, 0]


def _t3(tx_x, w1a, b1, aggf, w2t, b2, w3, b3):
    grid = (NTX // _T3B,)
    return pl.pallas_call(
        _t3_body,
        grid=grid,
        in_specs=[
            pl.BlockSpec((_T3B, F), lambda i: (i, 0)),
            pl.BlockSpec((F, D), lambda i: (0, 0)),
            pl.BlockSpec((1, D), lambda i: (0, 0)),
            pl.BlockSpec((_T3B, D), lambda i: (jnp.minimum(i, N // _T3B - 1), 0)),
            pl.BlockSpec((D, 64), lambda i: (0, 0)),
            pl.BlockSpec((1, 64), lambda i: (0, 0)),
            pl.BlockSpec((1, 64), lambda i: (0, 0)),
            pl.BlockSpec((1, 1), lambda i: (0, 0)),
        ],
        out_specs=pl.BlockSpec((_T3B, 1), lambda i: (i, 0)),
        out_shape=jax.ShapeDtypeStruct((NTX, 1), jnp.float32),
    )(tx_x, w1a, b1, aggf, w2t, b2, w3, b3)


def kernel(tx_x, entity_x, edge_index, emb_tables, tc_Wq, tc_bq, tc_Wk, tc_bk,
           tc_Wv, tc_bv, tc_Ws, tc_bs, ln_g, ln_b, cls_W1, cls_b1, cls_W2,
           cls_b2, cls_W3, cls_b3):
    f32 = jnp.float32
    offs = (jnp.arange(T, dtype=jnp.int32) * N)[:, None]

    flat_idx = (entity_x + offs).reshape(-1)
    flat_idx = jnp.pad(flat_idx, (0, NT_PAD - NT))
    table = emb_tables.reshape(NT, D)

    gsrc = jnp.pad(edge_index[:, 0, :] + offs, ((0, 0), (0, EP - E))).reshape(-1)
    gdst = jnp.pad(edge_index[:, 1, :] + offs, ((0, 0), (0, EP - E))).reshape(-1)
    edst = jnp.pad(edge_index[:, 1, :], ((0, 0), (0, EP - E)),
                   constant_values=IGNORED).reshape(-1)

    h0 = _s1(table, flat_idx)

    wt = jnp.concatenate([tc_Wq, tc_Wk, tc_Wv, tc_Ws], axis=0).T
    bcat = jnp.concatenate([tc_bq, tc_bk, tc_bv, tc_bs])[None]
    q, k, v2, sres = _t1(h0, wt, bcat)

    ex2 = _s2(q, k, gdst, gsrc)

    zrow = jnp.zeros((ZROWS, 64), f32)
    zden = jnp.zeros((ROWS_PER_TILE, 16), f32)
    numer, den = _s3(v2, gsrc, edst, ex2, zrow, zden)

    sres_t = sres[:NT].reshape(T, N, D)
    w1t = cls_W1[:, F:].reshape(D, T, D).transpose(1, 2, 0)
    p2 = _t2(numer, den, sres_t, ln_g[:, None], ln_b[:, None], w1t)

    p2f = p2.reshape(NC, NT, 64)
    agg = _s4(p2f, gsrc, edst, zrow)
    aggf = jnp.concatenate([agg[0], agg[1]], axis=-1)

    return _t3(tx_x, cls_W1[:, :F].T, cls_b1[None], aggf, cls_W2.T,
               cls_b2[None], cls_W3, cls_b3[None])


# speedup vs baseline: 3.4959x; 1.7234x over previous
import functools

import jax
import jax.numpy as jnp
from jax import lax
from jax.experimental import pallas as pl
from jax.experimental.pallas import tpu as pltpu
from jax.experimental.pallas import tpu_sc as plsc

T = 11
N = 20000
NT = T * N
E = 50000
D = 128
F = 394
NTX = 50000

NC = 2
NS = 16
L = 16
NW = NC * NS

S1_CHUNK = 128
S1_PER_W = 6912
NT_PAD = NW * S1_PER_W

EP = 51200
ETP = T * EP
S2_PER_W = ETP // NW
S2_CHUNK = 160
S3_PER_TILE = EP // NS
S3_CHUNK = 160
S4_PER_TILE = ETP // NS
S4_CHUNK = 160
ROWS_PER_TILE = N // NS
ZROWS = 125
IGNORED = N

_MESH = plsc.VectorSubcoreMesh(core_axis_name="c", subcore_axis_name="s")


def _s1_body(table, idx_hbm, out_hbm, idxb, rows, sem):
    c = lax.axis_index("c")
    s = lax.axis_index("s")
    wid = s * NC + c

    def chunk(i, carry):
        base = wid * S1_PER_W + i * S1_CHUNK
        pltpu.sync_copy(idx_hbm.at[pl.ds(base, S1_CHUNK)], idxb)
        pltpu.async_copy(table.at[idxb], rows, sem).wait()
        pltpu.sync_copy(rows, out_hbm.at[pl.ds(base, S1_CHUNK), :])
        return carry

    lax.fori_loop(0, S1_PER_W // S1_CHUNK, chunk, 0)


_s1 = pl.kernel(
    _s1_body,
    out_type=jax.ShapeDtypeStruct((NT_PAD, D), jnp.float32),
    mesh=_MESH,
    compiler_params=pltpu.CompilerParams(
        needs_layout_passes=False, use_tc_tiling_on_sc=False),
    scratch_types=[
        pltpu.VMEM((S1_CHUNK,), jnp.int32),
        pltpu.VMEM((S1_CHUNK, D), jnp.float32),
        pltpu.SemaphoreType.DMA,
    ],
)


def _s2_body(q_hbm, k_hbm, qidx_hbm, kidx_hbm, ex_hbm, qib, kib, qb, kb, exb,
             mini, sem):
    c = lax.axis_index("c")
    s = lax.axis_index("s")
    wid = s * NC + c
    inv_sqrt_d = 1.0 / (float(D) ** 0.5)
    lanes = lax.iota(jnp.int32, L)
    zc = jnp.zeros((L,), jnp.int32)

    def chunk(i, carry):
        base = wid * S2_PER_W + i * S2_CHUNK
        pltpu.sync_copy(qidx_hbm.at[pl.ds(base, S2_CHUNK)], qib)
        pltpu.sync_copy(kidx_hbm.at[pl.ds(base, S2_CHUNK)], kib)
        pltpu.async_copy(q_hbm.at[qib], qb, sem).wait()
        pltpu.async_copy(k_hbm.at[kib], kb, sem).wait()

        def group(g, carry2):
            e0 = g * L
            for e in range(L):
                rowv = zc + (e0 + e)
                acc = jnp.zeros((L,), jnp.float32)
                for j in range(D // L):
                    cols = lanes + j * L
                    qv = plsc.load_gather(qb, [rowv, cols])
                    kv = plsc.load_gather(kb, [rowv, cols])
                    acc = acc + qv * kv
                plsc.store_scatter(mini, [jnp.full((L,), e, jnp.int32), lanes],
                                   acc)
            tot = jnp.zeros((L,), jnp.float32)
            for ccol in range(L):
                tot = tot + plsc.load_gather(
                    mini, [lanes, jnp.full((L,), ccol, jnp.int32)])
            exv = jnp.exp(tot * inv_sqrt_d)
            plsc.store_scatter(exb, [lanes + e0, zc], exv)
            return carry2

        lax.fori_loop(0, S2_CHUNK // L, group, 0)
        pltpu.sync_copy(exb, ex_hbm.at[pl.ds(base, S2_CHUNK), :])
        return carry

    lax.fori_loop(0, S2_PER_W // S2_CHUNK, chunk, 0)


_s2 = pl.kernel(
    _s2_body,
    out_type=jax.ShapeDtypeStruct((ETP, 1), jnp.float32),
    mesh=_MESH,
    compiler_params=pltpu.CompilerParams(
        needs_layout_passes=False, use_tc_tiling_on_sc=False),
    scratch_types=[
        pltpu.VMEM((S2_CHUNK,), jnp.int32),
        pltpu.VMEM((S2_CHUNK,), jnp.int32),
        pltpu.VMEM((S2_CHUNK, D), jnp.float32),
        pltpu.VMEM((S2_CHUNK, D), jnp.float32),
        pltpu.VMEM((S2_CHUNK, 1), jnp.float32),
        pltpu.VMEM((L, 17), jnp.float32),
        pltpu.SemaphoreType.DMA,
    ],
)


def _s3_body(v2, esrc, edst, ex2, zrow, zden, numer, den,
             srcb, dstb, exb, exb16, vb, zb, accum, dacc, sem):
    c = lax.axis_index("c")
    s = lax.axis_index("s")
    pltpu.sync_copy(zrow, zb)
    pltpu.sync_copy(zden.at[pl.ds(0, S3_CHUNK), :], exb16)
    row0 = s * ROWS_PER_TILE

    def per_type(t, carry):
        plsc.subcore_barrier()
        for j in range(ROWS_PER_TILE // ZROWS):
            pltpu.sync_copy(zb, accum.at[pl.ds(row0 + j * ZROWS, ZROWS), :])

        @pl.when(c == 0)
        def _():
            pltpu.sync_copy(zden, dacc.at[pl.ds(row0, ROWS_PER_TILE), :])

        plsc.subcore_barrier()

        def chunk(i, carry2):
            base = t * EP + s * S3_PER_TILE + i * S3_CHUNK
            pltpu.sync_copy(esrc.at[pl.ds(base, S3_CHUNK)], srcb)
            pltpu.sync_copy(edst.at[pl.ds(base, S3_CHUNK)], dstb)
            pltpu.sync_copy(ex2.at[pl.ds(base, S3_CHUNK), :], exb)
            pltpu.async_copy(v2.at[c].at[srcb], vb, sem).wait()
            lanes = lax.iota(jnp.int32, L)
            zc = jnp.zeros((L,), jnp.int32)

            @pl.when(c == 0)
            def _():
                def g_body(g, carry3):
                    rows = lanes + g * L
                    exv = plsc.load_gather(exb, [rows, zc])
                    plsc.store_scatter(exb16, [rows, zc], exv)
                    return carry3

                lax.fori_loop(0, S3_CHUNK // L, g_body, 0)

            def edge_body(e, carry3):
                rowv = zc + e
                exsplat = plsc.load_gather(exb, [rowv, zc])
                for j in range(64 // L):
                    cols = lanes + j * L
                    vv = plsc.load_gather(vb, [rowv, cols])
                    plsc.store_scatter(vb, [rowv, cols], vv * exsplat)
                return carry3

            lax.fori_loop(0, S3_CHUNK, edge_body, 0, unroll=2)
            pltpu.sync_copy(
                vb, accum.at[plsc.Indices(dstb, ignored_value=IGNORED)],
                add=True)

            @pl.when(c == 0)
            def _():
                pltpu.sync_copy(
                    exb16, dacc.at[plsc.Indices(dstb, ignored_value=IGNORED)],
                    add=True)

            return carry2

        lax.fori_loop(0, S3_PER_TILE // S3_CHUNK, chunk, 0)
        plsc.subcore_barrier()
        pltpu.sync_copy(accum.at[pl.ds(row0, ROWS_PER_TILE), :],
                        numer.at[c, t, pl.ds(row0, ROWS_PER_TILE), :])

        @pl.when(c == 0)
        def _():
            pltpu.sync_copy(dacc.at[pl.ds(row0, ROWS_PER_TILE), :],
                            den.at[t, pl.ds(row0, ROWS_PER_TILE), :])

        return carry

    lax.fori_loop(0, T, per_type, 0)


_s3 = pl.kernel(
    _s3_body,
    out_type=(
        jax.ShapeDtypeStruct((NC, T, N, 64), jnp.float32),
        jax.ShapeDtypeStruct((T, N, 16), jnp.float32),
    ),
    mesh=_MESH,
    compiler_params=pltpu.CompilerParams(
        needs_layout_passes=False, use_tc_tiling_on_sc=False),
    scratch_types=[
        pltpu.VMEM((S3_CHUNK,), jnp.int32),
        pltpu.VMEM((S3_CHUNK,), jnp.int32),
        pltpu.VMEM((S3_CHUNK, 1), jnp.float32),
        pltpu.VMEM((S3_CHUNK, 16), jnp.float32),
        pltpu.VMEM((S3_CHUNK, 64), jnp.float32),
        pltpu.VMEM((ZROWS, 64), jnp.float32),
        pltpu.VMEM_SHARED((N, 64), jnp.float32),
        pltpu.VMEM_SHARED((N, 16), jnp.float32),
        pltpu.SemaphoreType.DMA,
    ],
)


def _s4_body(p2, esrc, edst, zrow, agg, srcb, dstb, pb, zb, accum, sem):
    c = lax.axis_index("c")
    s = lax.axis_index("s")
    pltpu.sync_copy(zrow, zb)
    row0 = s * ROWS_PER_TILE
    for j in range(ROWS_PER_TILE // ZROWS):
        pltpu.sync_copy(zb, accum.at[pl.ds(row0 + j * ZROWS, ZROWS), :])
    plsc.subcore_barrier()

    def chunk(i, carry):
        base = s * S4_PER_TILE + i * S4_CHUNK
        pltpu.sync_copy(esrc.at[pl.ds(base, S4_CHUNK)], srcb)
        pltpu.sync_copy(edst.at[pl.ds(base, S4_CHUNK)], dstb)
        pltpu.async_copy(p2.at[c].at[srcb], pb, sem).wait()
        pltpu.sync_copy(
            pb, accum.at[plsc.Indices(dstb, ignored_value=IGNORED)], add=True)
        return carry

    lax.fori_loop(0, S4_PER_TILE // S4_CHUNK, chunk, 0)
    plsc.subcore_barrier()
    pltpu.sync_copy(accum.at[pl.ds(row0, ROWS_PER_TILE), :],
                    agg.at[c, pl.ds(row0, ROWS_PER_TILE), :])


_s4 = pl.kernel(
    _s4_body,
    out_type=jax.ShapeDtypeStruct((NC, N, 64), jnp.float32),
    mesh=_MESH,
    compiler_params=pltpu.CompilerParams(
        needs_layout_passes=False, use_tc_tiling_on_sc=False),
    scratch_types=[
        pltpu.VMEM((S4_CHUNK,), jnp.int32),
        pltpu.VMEM((S4_CHUNK,), jnp.int32),
        pltpu.VMEM((S4_CHUNK, 64), jnp.float32),
        pltpu.VMEM((ZROWS, 64), jnp.float32),
        pltpu.VMEM_SHARED((N, 64), jnp.float32),
        pltpu.SemaphoreType.DMA,
    ],
)


_T1B = 2048


def _t1_body(x_ref, w_ref, b_ref, q_ref, k_ref, v2_ref, s_ref):
    y = jnp.dot(x_ref[...], w_ref[...],
                preferred_element_type=jnp.float32) + b_ref[...]
    q_ref[...] = y[:, :128]
    k_ref[...] = y[:, 128:256]
    v2_ref[0] = y[:, 256:320]
    v2_ref[1] = y[:, 320:384]
    s_ref[...] = y[:, 384:]


def _t1(h0, wt, bcat):
    grid = (NT_PAD // _T1B,)
    return pl.pallas_call(
        _t1_body,
        grid=grid,
        in_specs=[
            pl.BlockSpec((_T1B, D), lambda i: (i, 0)),
            pl.BlockSpec((D, 512), lambda i: (0, 0)),
            pl.BlockSpec((1, 512), lambda i: (0, 0)),
        ],
        out_specs=[
            pl.BlockSpec((_T1B, D), lambda i: (i, 0)),
            pl.BlockSpec((_T1B, D), lambda i: (i, 0)),
            pl.BlockSpec((NC, _T1B, 64), lambda i: (0, i, 0)),
            pl.BlockSpec((_T1B, D), lambda i: (i, 0)),
        ],
        out_shape=[
            jax.ShapeDtypeStruct((NT_PAD, D), jnp.float32),
            jax.ShapeDtypeStruct((NT_PAD, D), jnp.float32),
            jax.ShapeDtypeStruct((NC, NT_PAD, 64), jnp.float32),
            jax.ShapeDtypeStruct((NT_PAD, D), jnp.float32),
        ],
    )(h0, wt, bcat)


_T2B = 2000


def _t2_body(n_ref, d_ref, s_ref, g_ref, b_ref, w_ref, p_ref):
    nl = n_ref[0, 0]
    nh = n_ref[1, 0]
    den = d_ref[0, :, :1] + 1e-16
    h = jnp.concatenate([nl, nh], axis=-1) / den + s_ref[0]
    mu = jnp.mean(h, axis=-1, keepdims=True)
    r = h - mu
    var = jnp.mean(r * r, axis=-1, keepdims=True)
    hn = r / jnp.sqrt(var + 1e-5) * g_ref[0, 0] + b_ref[0, 0]
    p = jnp.dot(hn, w_ref[0], preferred_element_type=jnp.float32)
    p_ref[0, 0] = p[:, :64]
    p_ref[1, 0] = p[:, 64:]


def _t2(numer, den, sres, g, b, w1t):
    grid = (T, N // _T2B)
    return pl.pallas_call(
        _t2_body,
        grid=grid,
        in_specs=[
            pl.BlockSpec((NC, 1, _T2B, 64), lambda t, r: (0, t, r, 0)),
            pl.BlockSpec((1, _T2B, 16), lambda t, r: (t, r, 0)),
            pl.BlockSpec((1, _T2B, D), lambda t, r: (t, r, 0)),
            pl.BlockSpec((1, 1, D), lambda t, r: (t, 0, 0)),
            pl.BlockSpec((1, 1, D), lambda t, r: (t, 0, 0)),
            pl.BlockSpec((1, D, D), lambda t, r: (t, 0, 0)),
        ],
        out_specs=pl.BlockSpec((NC, 1, _T2B, 64), lambda t, r: (0, t, r, 0)),
        out_shape=jax.ShapeDtypeStruct((NC, T, N, 64), jnp.float32),
    )(numer, den, sres, g, b, w1t)


_T3B = 2000


def _t3_body(x_ref, w1_ref, b1_ref, a_ref, w2_ref, b2_ref, w3_ref, b3_ref,
             o_ref):
    i = pl.program_id(0)
    acc = jnp.dot(x_ref[...], w1_ref[...],
                  preferred_element_type=jnp.float32) + b1_ref[...]
    acc = acc + jnp.where(i < N // _T3B, 1.0, 0.0) * a_ref[...]
    z = jnp.maximum(acc, 0.0)
    z2 = jnp.maximum(
        jnp.dot(z, w2_ref[...], preferred_element_type=jnp.float32)
        + b2_ref[...], 0.0)
    o_ref[...] = jnp.sum(z2 * w3_ref[...], axis=-1, keepdims=True) + b3_ref[0, 0]


def _t3(tx_x, w1a, b1, aggf, w2t, b2, w3, b3):
    grid = (NTX // _T3B,)
    return pl.pallas_call(
        _t3_body,
        grid=grid,
        in_specs=[
            pl.BlockSpec((_T3B, F), lambda i: (i, 0)),
            pl.BlockSpec((F, D), lambda i: (0, 0)),
            pl.BlockSpec((1, D), lambda i: (0, 0)),
            pl.BlockSpec((_T3B, D), lambda i: (jnp.minimum(i, N // _T3B - 1), 0)),
            pl.BlockSpec((D, 64), lambda i: (0, 0)),
            pl.BlockSpec((1, 64), lambda i: (0, 0)),
            pl.BlockSpec((1, 64), lambda i: (0, 0)),
            pl.BlockSpec((1, 1), lambda i: (0, 0)),
        ],
        out_specs=pl.BlockSpec((_T3B, 1), lambda i: (i, 0)),
        out_shape=jax.ShapeDtypeStruct((NTX, 1), jnp.float32),
    )(tx_x, w1a, b1, aggf, w2t, b2, w3, b3)


def kernel(tx_x, entity_x, edge_index, emb_tables, tc_Wq, tc_bq, tc_Wk, tc_bk,
           tc_Wv, tc_bv, tc_Ws, tc_bs, ln_g, ln_b, cls_W1, cls_b1, cls_W2,
           cls_b2, cls_W3, cls_b3):
    f32 = jnp.float32
    offs = (jnp.arange(T, dtype=jnp.int32) * N)[:, None]

    flat_idx = (entity_x + offs).reshape(-1)
    flat_idx = jnp.pad(flat_idx, (0, NT_PAD - NT))
    table = emb_tables.reshape(NT, D)

    gsrc = jnp.pad(edge_index[:, 0, :] + offs, ((0, 0), (0, EP - E))).reshape(-1)
    gdst = jnp.pad(edge_index[:, 1, :] + offs, ((0, 0), (0, EP - E))).reshape(-1)
    edst = jnp.pad(edge_index[:, 1, :], ((0, 0), (0, EP - E)),
                   constant_values=IGNORED).reshape(-1)

    h0 = _s1(table, flat_idx)

    wt = jnp.concatenate([tc_Wq, tc_Wk, tc_Wv, tc_Ws], axis=0).T
    bcat = jnp.concatenate([tc_bq, tc_bk, tc_bv, tc_bs])[None]
    q, k, v2, sres = _t1(h0, wt, bcat)

    ex2 = _s2(q, k, gdst, gsrc)

    zrow = jnp.zeros((ZROWS, 64), f32)
    zden = jnp.zeros((ROWS_PER_TILE, 16), f32)
    numer, den = _s3(v2, gsrc, edst, ex2, zrow, zden)

    sres_t = sres[:NT].reshape(T, N, D)
    w1t = cls_W1[:, F:].reshape(D, T, D).transpose(1, 2, 0)
    p2 = _t2(numer, den, sres_t, ln_g[:, None], ln_b[:, None], w1t)

    p2f = p2.reshape(NC, NT, 64)
    agg = _s4(p2f, gsrc, edst, zrow)
    aggf = jnp.concatenate([agg[0], agg[1]], axis=-1)

    return _t3(tx_x, cls_W1[:, :F].T, cls_b1[None], aggf, cls_W2.T,
               cls_b2[None], cls_W3, cls_b3[None])


# speedup vs baseline: 4.1908x; 1.1988x over previous
import functools

import jax
import jax.numpy as jnp
from jax import lax
from jax.experimental import pallas as pl
from jax.experimental.pallas import tpu as pltpu
from jax.experimental.pallas import tpu_sc as plsc

T = 11
N = 20000
NT = T * N
E = 50000
D = 128
F = 394
NTX = 50000

NC = 2
NS = 16
L = 16
NW = NC * NS

S1_CHUNK = 128
S1_PER_W = 6912
NT_PAD = NW * S1_PER_W

EP = 51200
ETP = T * EP
S2_PER_W = ETP // NW
S2_CHUNK = 160
S3_PER_TILE = EP // NS
S3_CHUNK = 160
S4_PER_TILE = ETP // NS
S4_CHUNK = 160
ROWS_PER_TILE = N // NS
ZROWS = 125
IGNORED = N

_MESH = plsc.VectorSubcoreMesh(core_axis_name="c", subcore_axis_name="s")


def _s1_body(table, idx_hbm, out_hbm, idxb, rows, sem):
    c = lax.axis_index("c")
    s = lax.axis_index("s")
    wid = s * NC + c

    def chunk(i, carry):
        base = wid * S1_PER_W + i * S1_CHUNK
        pltpu.sync_copy(idx_hbm.at[pl.ds(base, S1_CHUNK)], idxb)
        pltpu.async_copy(table.at[idxb], rows, sem).wait()
        pltpu.sync_copy(rows, out_hbm.at[pl.ds(base, S1_CHUNK), :])
        return carry

    lax.fori_loop(0, S1_PER_W // S1_CHUNK, chunk, 0)


_s1 = pl.kernel(
    _s1_body,
    out_type=jax.ShapeDtypeStruct((NT_PAD, D), jnp.float32),
    mesh=_MESH,
    compiler_params=pltpu.CompilerParams(
        needs_layout_passes=False, use_tc_tiling_on_sc=False),
    scratch_types=[
        pltpu.VMEM((S1_CHUNK,), jnp.int32),
        pltpu.VMEM((S1_CHUNK, D), jnp.float32),
        pltpu.SemaphoreType.DMA,
    ],
)


def _s2_body(q_hbm, k_hbm, qidx_hbm, kidx_hbm, ex_hbm, qiv, kiv,
             qb0, qb1, kb0, kb1, exb0, exb1, mini, semg0, semg1, semx):
    c = lax.axis_index("c")
    s = lax.axis_index("s")
    wid = s * NC + c
    inv_sqrt_d = 1.0 / (float(D) ** 0.5)
    lanes = lax.iota(jnp.int32, L)
    zc = jnp.zeros((L,), jnp.int32)
    tbase = wid * S2_PER_W
    NCH = S2_PER_W // S2_CHUNK
    qbs = (qb0, qb1)
    kbs = (kb0, kb1)
    exbs = (exb0, exb1)
    semg = (semg0, semg1)

    pltpu.sync_copy(qidx_hbm.at[pl.ds(tbase, S2_PER_W)], qiv)
    pltpu.sync_copy(kidx_hbm.at[pl.ds(tbase, S2_PER_W)], kiv)

    def issue(i, b):
        pltpu.async_copy(q_hbm.at[qiv.at[pl.ds(i * S2_CHUNK, S2_CHUNK)]],
                         qbs[b], semg[b])
        pltpu.async_copy(k_hbm.at[kiv.at[pl.ds(i * S2_CHUNK, S2_CHUNK)]],
                         kbs[b], semg[b])

    issue(0, 0)
    issue(1, 1)

    def outer(i2, carry):
        for b in range(2):
            i = i2 * 2 + b
            base = tbase + i * S2_CHUNK

            @pl.when(i >= 2)
            def _():
                pltpu.make_async_copy(
                    exbs[b], ex_hbm.at[pl.ds(tbase, S2_CHUNK), :], semx).wait()

            pltpu.make_async_copy(q_hbm.at[pl.ds(0, S2_CHUNK), :], qbs[b],
                                  semg[b]).wait()
            pltpu.make_async_copy(k_hbm.at[pl.ds(0, S2_CHUNK), :], kbs[b],
                                  semg[b]).wait()

            def group(g, carry2):
                e0 = g * L
                for e in range(L):
                    rowv = zc + (e0 + e)
                    acc = jnp.zeros((L,), jnp.float32)
                    for j in range(D // L):
                        cols = lanes + j * L
                        qv = plsc.load_gather(qbs[b], [rowv, cols])
                        kv = plsc.load_gather(kbs[b], [rowv, cols])
                        acc = acc + qv * kv
                    plsc.store_scatter(
                        mini, [jnp.full((L,), e, jnp.int32), lanes], acc)
                tot = jnp.zeros((L,), jnp.float32)
                for ccol in range(L):
                    tot = tot + plsc.load_gather(
                        mini, [lanes, jnp.full((L,), ccol, jnp.int32)])
                exv = jnp.exp(tot * inv_sqrt_d)
                plsc.store_scatter(exbs[b], [lanes + e0, zc], exv)
                return carry2

            lax.fori_loop(0, S2_CHUNK // L, group, 0)
            pltpu.async_copy(exbs[b], ex_hbm.at[pl.ds(base, S2_CHUNK), :],
                             semx)

            @pl.when(i + 2 < NCH)
            def _():
                issue(i + 2, b)

        return carry

    lax.fori_loop(0, NCH // 2, outer, 0)
    for b in range(2):
        pltpu.make_async_copy(
            exbs[b], ex_hbm.at[pl.ds(tbase, S2_CHUNK), :], semx).wait()


_s2 = pl.kernel(
    _s2_body,
    out_type=jax.ShapeDtypeStruct((ETP, 1), jnp.float32),
    mesh=_MESH,
    compiler_params=pltpu.CompilerParams(
        needs_layout_passes=False, use_tc_tiling_on_sc=False),
    scratch_types=[
        pltpu.VMEM((S2_PER_W,), jnp.int32),
        pltpu.VMEM((S2_PER_W,), jnp.int32),
        pltpu.VMEM((S2_CHUNK, D), jnp.float32),
        pltpu.VMEM((S2_CHUNK, D), jnp.float32),
        pltpu.VMEM((S2_CHUNK, D), jnp.float32),
        pltpu.VMEM((S2_CHUNK, D), jnp.float32),
        pltpu.VMEM((S2_CHUNK, 1), jnp.float32),
        pltpu.VMEM((S2_CHUNK, 1), jnp.float32),
        pltpu.VMEM((L, 17), jnp.float32),
        pltpu.SemaphoreType.DMA,
        pltpu.SemaphoreType.DMA,
        pltpu.SemaphoreType.DMA,
    ],
)


def _s3_body(v2, esrc, edst2, ex2, zrow, zden, numer, den,
             siv, div, exv_pre, vb0, vb1, eb0, eb1,
             accum, dacc, semg0, semg1, sv0, sv1, se0, se1):
    c = lax.axis_index("c")
    s = lax.axis_index("s")
    row0 = s * ROWS_PER_TILE
    lanes = lax.iota(jnp.int32, L)
    zc = jnp.zeros((L,), jnp.int32)
    vbs = (vb0, vb1)
    ebs = (eb0, eb1)
    semg = (semg0, semg1)
    semv = (sv0, sv1)
    seme = (se0, se1)
    NCH = S3_PER_TILE // S3_CHUNK

    for b in range(2):
        pltpu.sync_copy(zden.at[pl.ds(0, S3_CHUNK), :], ebs[b])

    def issue_gather(j, b):
        pltpu.async_copy(
            v2.at[c].at[siv.at[pl.ds(j * S3_CHUNK, S3_CHUNK)]], vbs[b],
            semg[b])

    def per_type(t, carry):
        plsc.subcore_barrier()
        pltpu.sync_copy(zrow, accum.at[pl.ds(row0, ROWS_PER_TILE), :])

        @pl.when(c == 0)
        def _():
            pltpu.sync_copy(zden, dacc.at[pl.ds(row0, ROWS_PER_TILE), :])

        tb = t * EP + s * S3_PER_TILE
        pltpu.sync_copy(esrc.at[pl.ds(tb, S3_PER_TILE)], siv)
        pltpu.sync_copy(
            edst2.at[pl.ds(t * (EP // S3_CHUNK) + s * NCH, NCH), :], div)
        pltpu.sync_copy(ex2.at[pl.ds(tb, S3_PER_TILE)], exv_pre)
        plsc.subcore_barrier()
        issue_gather(0, 0)
        issue_gather(1, 1)

        def outer(j2, carry2):
            for b in range(2):
                j = j2 * 2 + b
                cb = j * S3_CHUNK
                pltpu.make_async_copy(
                    v2.at[c].at[pl.ds(0, S3_CHUNK)], vbs[b], semg[b]).wait()

                @pl.when(c == 0)
                def _():
                    def g_body(g, carry3):
                        rows = lanes + g * L
                        exv = plsc.load_gather(exv_pre, [rows + cb])
                        plsc.store_scatter(ebs[b], [rows, zc], exv)
                        return carry3

                    lax.fori_loop(0, S3_CHUNK // L, g_body, 0)

                def edge_body(e, carry3):
                    rowv = zc + e
                    exsplat = plsc.load_gather(exv_pre, [zc + (cb + e)])
                    for j4 in range(64 // L):
                        cols = lanes + j4 * L
                        vv = plsc.load_gather(vbs[b], [rowv, cols])
                        plsc.store_scatter(vbs[b], [rowv, cols], vv * exsplat)
                    return carry3

                lax.fori_loop(0, S3_CHUNK, edge_body, 0, unroll=2)
                pltpu.async_copy(
                    vbs[b],
                    accum.at[plsc.Indices(div.at[j], ignored_value=IGNORED)],
                    semv[b], add=True)

                @pl.when(c == 0)
                def _():
                    pltpu.async_copy(
                        ebs[b],
                        dacc.at[plsc.Indices(div.at[j],
                                             ignored_value=IGNORED)],
                        seme[b], add=True)

                @pl.when(j + 2 < NCH)
                def _():
                    pltpu.make_async_copy(
                        vbs[b], accum.at[pl.ds(0, S3_CHUNK), :],
                        semv[b]).wait()

                    @pl.when(c == 0)
                    def _():
                        pltpu.make_async_copy(
                            ebs[b], dacc.at[pl.ds(0, S3_CHUNK), :],
                            seme[b]).wait()

                    issue_gather(j + 2, b)

            return carry2

        lax.fori_loop(0, NCH // 2, outer, 0)
        for b in range(2):
            pltpu.make_async_copy(
                vbs[b], accum.at[pl.ds(0, S3_CHUNK), :], semv[b]).wait()

            @pl.when(c == 0)
            def _():
                pltpu.make_async_copy(
                    ebs[b], dacc.at[pl.ds(0, S3_CHUNK), :], seme[b]).wait()

        plsc.subcore_barrier()
        pltpu.sync_copy(accum.at[pl.ds(row0, ROWS_PER_TILE), :],
                        numer.at[c, t, pl.ds(row0, ROWS_PER_TILE), :])

        @pl.when(c == 0)
        def _():
            pltpu.sync_copy(dacc.at[pl.ds(row0, ROWS_PER_TILE), :],
                            den.at[t, pl.ds(row0, ROWS_PER_TILE), :])

        return carry

    lax.fori_loop(0, T, per_type, 0)


_s3 = pl.kernel(
    _s3_body,
    out_type=(
        jax.ShapeDtypeStruct((NC, T, N, 64), jnp.float32),
        jax.ShapeDtypeStruct((T, N, 8), jnp.float32),
    ),
    mesh=_MESH,
    compiler_params=pltpu.CompilerParams(
        needs_layout_passes=False, use_tc_tiling_on_sc=False),
    scratch_types=[
        pltpu.VMEM((S3_PER_TILE,), jnp.int32),
        pltpu.VMEM((S3_PER_TILE // S3_CHUNK, S3_CHUNK), jnp.int32),
        pltpu.VMEM((S3_PER_TILE,), jnp.float32),
        pltpu.VMEM((S3_CHUNK, 64), jnp.float32),
        pltpu.VMEM((S3_CHUNK, 64), jnp.float32),
        pltpu.VMEM((S3_CHUNK, 8), jnp.float32),
        pltpu.VMEM((S3_CHUNK, 8), jnp.float32),
        pltpu.VMEM_SHARED((N, 64), jnp.float32),
        pltpu.VMEM_SHARED((N, 8), jnp.float32),
        pltpu.SemaphoreType.DMA,
        pltpu.SemaphoreType.DMA,
        pltpu.SemaphoreType.DMA,
        pltpu.SemaphoreType.DMA,
        pltpu.SemaphoreType.DMA,
        pltpu.SemaphoreType.DMA,
    ],
)


def _s4_body(p2, esrc, edst, zrow, agg, srcb, dstb, pb, accum, sem):
    c = lax.axis_index("c")
    s = lax.axis_index("s")
    row0 = s * ROWS_PER_TILE
    pltpu.sync_copy(zrow, accum.at[pl.ds(row0, ROWS_PER_TILE), :])
    plsc.subcore_barrier()

    def chunk(i, carry):
        base = s * S4_PER_TILE + i * S4_CHUNK
        pltpu.sync_copy(esrc.at[pl.ds(base, S4_CHUNK)], srcb)
        pltpu.sync_copy(edst.at[pl.ds(base, S4_CHUNK)], dstb)
        pltpu.async_copy(p2.at[c].at[srcb], pb, sem).wait()
        pltpu.sync_copy(
            pb, accum.at[plsc.Indices(dstb, ignored_value=IGNORED)], add=True)
        return carry

    lax.fori_loop(0, S4_PER_TILE // S4_CHUNK, chunk, 0)
    plsc.subcore_barrier()
    pltpu.sync_copy(accum.at[pl.ds(row0, ROWS_PER_TILE), :],
                    agg.at[c, pl.ds(row0, ROWS_PER_TILE), :])


_s4 = pl.kernel(
    _s4_body,
    out_type=jax.ShapeDtypeStruct((NC, N, 64), jnp.float32),
    mesh=_MESH,
    compiler_params=pltpu.CompilerParams(
        needs_layout_passes=False, use_tc_tiling_on_sc=False),
    scratch_types=[
        pltpu.VMEM((S4_CHUNK,), jnp.int32),
        pltpu.VMEM((S4_CHUNK,), jnp.int32),
        pltpu.VMEM((S4_CHUNK, 64), jnp.float32),
        pltpu.VMEM_SHARED((N, 64), jnp.float32),
        pltpu.SemaphoreType.DMA,
    ],
)


_T1B = 2048


def _t1_body(x_ref, w_ref, b_ref, q_ref, k_ref, v2_ref, s_ref):
    y = jnp.dot(x_ref[...], w_ref[...],
                preferred_element_type=jnp.float32) + b_ref[...]
    q_ref[...] = y[:, :128]
    k_ref[...] = y[:, 128:256]
    v2_ref[0] = y[:, 256:320]
    v2_ref[1] = y[:, 320:384]
    s_ref[...] = y[:, 384:]


def _t1(h0, wt, bcat):
    grid = (NT_PAD // _T1B,)
    return pl.pallas_call(
        _t1_body,
        grid=grid,
        in_specs=[
            pl.BlockSpec((_T1B, D), lambda i: (i, 0)),
            pl.BlockSpec((D, 512), lambda i: (0, 0)),
            pl.BlockSpec((1, 512), lambda i: (0, 0)),
        ],
        out_specs=[
            pl.BlockSpec((_T1B, D), lambda i: (i, 0)),
            pl.BlockSpec((_T1B, D), lambda i: (i, 0)),
            pl.BlockSpec((NC, _T1B, 64), lambda i: (0, i, 0)),
            pl.BlockSpec((_T1B, D), lambda i: (i, 0)),
        ],
        out_shape=[
            jax.ShapeDtypeStruct((NT_PAD, D), jnp.float32),
            jax.ShapeDtypeStruct((NT_PAD, D), jnp.float32),
            jax.ShapeDtypeStruct((NC, NT_PAD, 64), jnp.float32),
            jax.ShapeDtypeStruct((NT_PAD, D), jnp.float32),
        ],
    )(h0, wt, bcat)


_T2B = 2000


def _t2_body(n_ref, d_ref, s_ref, g_ref, b_ref, w_ref, p_ref):
    nl = n_ref[0, 0]
    nh = n_ref[1, 0]
    den = d_ref[0, :, :1] + 1e-16
    h = jnp.concatenate([nl, nh], axis=-1) / den + s_ref[0]
    mu = jnp.mean(h, axis=-1, keepdims=True)
    r = h - mu
    var = jnp.mean(r * r, axis=-1, keepdims=True)
    hn = r / jnp.sqrt(var + 1e-5) * g_ref[0, 0] + b_ref[0, 0]
    p = jnp.dot(hn, w_ref[0], preferred_element_type=jnp.float32)
    p_ref[0, 0] = p[:, :64]
    p_ref[1, 0] = p[:, 64:]


def _t2(numer, den, sres, g, b, w1t):
    grid = (T, N // _T2B)
    return pl.pallas_call(
        _t2_body,
        grid=grid,
        in_specs=[
            pl.BlockSpec((NC, 1, _T2B, 64), lambda t, r: (0, t, r, 0)),
            pl.BlockSpec((1, _T2B, 8), lambda t, r: (t, r, 0)),
            pl.BlockSpec((1, _T2B, D), lambda t, r: (t, r, 0)),
            pl.BlockSpec((1, 1, D), lambda t, r: (t, 0, 0)),
            pl.BlockSpec((1, 1, D), lambda t, r: (t, 0, 0)),
            pl.BlockSpec((1, D, D), lambda t, r: (t, 0, 0)),
        ],
        out_specs=pl.BlockSpec((NC, 1, _T2B, 64), lambda t, r: (0, t, r, 0)),
        out_shape=jax.ShapeDtypeStruct((NC, T, N, 64), jnp.float32),
    )(numer, den, sres, g, b, w1t)


_T3B = 2000


def _t3_body(x_ref, w1_ref, b1_ref, a_ref, w2_ref, b2_ref, w3_ref, b3_ref,
             o_ref):
    i = pl.program_id(0)
    acc = jnp.dot(x_ref[...], w1_ref[...],
                  preferred_element_type=jnp.float32) + b1_ref[...]
    acc = acc + jnp.where(i < N // _T3B, 1.0, 0.0) * a_ref[...]
    z = jnp.maximum(acc, 0.0)
    z2 = jnp.maximum(
        jnp.dot(z, w2_ref[...], preferred_element_type=jnp.float32)
        + b2_ref[...], 0.0)
    o_ref[...] = jnp.sum(z2 * w3_ref[...], axis=-1, keepdims=True) + b3_ref[0, 0]


def _t3(tx_x, w1a, b1, aggf, w2t, b2, w3, b3):
    grid = (NTX // _T3B,)
    return pl.pallas_call(
        _t3_body,
        grid=grid,
        in_specs=[
            pl.BlockSpec((_T3B, F), lambda i: (i, 0)),
            pl.BlockSpec((F, D), lambda i: (0, 0)),
            pl.BlockSpec((1, D), lambda i: (0, 0)),
            pl.BlockSpec((_T3B, D), lambda i: (jnp.minimum(i, N // _T3B - 1), 0)),
            pl.BlockSpec((D, 64), lambda i: (0, 0)),
            pl.BlockSpec((1, 64), lambda i: (0, 0)),
            pl.BlockSpec((1, 64), lambda i: (0, 0)),
            pl.BlockSpec((1, 1), lambda i: (0, 0)),
        ],
        out_specs=pl.BlockSpec((_T3B, 1), lambda i: (i, 0)),
        out_shape=jax.ShapeDtypeStruct((NTX, 1), jnp.float32),
    )(tx_x, w1a, b1, aggf, w2t, b2, w3, b3)


def kernel(tx_x, entity_x, edge_index, emb_tables, tc_Wq, tc_bq, tc_Wk, tc_bk,
           tc_Wv, tc_bv, tc_Ws, tc_bs, ln_g, ln_b, cls_W1, cls_b1, cls_W2,
           cls_b2, cls_W3, cls_b3):
    f32 = jnp.float32
    offs = (jnp.arange(T, dtype=jnp.int32) * N)[:, None]

    flat_idx = (entity_x + offs).reshape(-1)
    flat_idx = jnp.pad(flat_idx, (0, NT_PAD - NT))
    table = emb_tables.reshape(NT, D)

    gsrc = jnp.pad(edge_index[:, 0, :] + offs, ((0, 0), (0, EP - E))).reshape(-1)
    gdst = jnp.pad(edge_index[:, 1, :] + offs, ((0, 0), (0, EP - E))).reshape(-1)
    edst = jnp.pad(edge_index[:, 1, :], ((0, 0), (0, EP - E)),
                   constant_values=IGNORED).reshape(-1)

    h0 = _s1(table, flat_idx)

    wt = jnp.concatenate([tc_Wq, tc_Wk, tc_Wv, tc_Ws], axis=0).T
    bcat = jnp.concatenate([tc_bq, tc_bk, tc_bv, tc_bs])[None]
    q, k, v2, sres = _t1(h0, wt, bcat)

    ex2 = _s2(q, k, gdst, gsrc)

    zrow = jnp.zeros((ROWS_PER_TILE, 64), f32)
    zden = jnp.zeros((ROWS_PER_TILE, 8), f32)
    numer, den = _s3(v2, gsrc, edst.reshape(-1, S3_CHUNK), ex2.reshape(-1),
                     zrow, zden)

    sres_t = sres[:NT].reshape(T, N, D)
    w1t = cls_W1[:, F:].reshape(D, T, D).transpose(1, 2, 0)
    p2 = _t2(numer, den, sres_t, ln_g[:, None], ln_b[:, None], w1t)

    p2f = p2.reshape(NC, NT, 64)
    agg = _s4(p2f, gsrc, edst, zrow)
    aggf = jnp.concatenate([agg[0], agg[1]], axis=-1)

    return _t3(tx_x, cls_W1[:, :F].T, cls_b1[None], aggf, cls_W2.T,
               cls_b2[None], cls_W3, cls_b3[None])


# speedup vs baseline: 4.4760x; 1.0681x over previous
import functools

import jax
import jax.numpy as jnp
from jax import lax
from jax.experimental import pallas as pl
from jax.experimental.pallas import tpu as pltpu
from jax.experimental.pallas import tpu_sc as plsc

T = 11
N = 20000
NT = T * N
E = 50000
D = 128
F = 394
NTX = 50000

NC = 2
NS = 16
L = 16
NW = NC * NS

S1_CHUNK = 128
S1_PER_W = 6912
NT_PAD = NW * S1_PER_W

EP = 51200
ETP = T * EP
S2_PER_W = ETP // NW
S2_CHUNK = 160
S3_PER_TILE = EP // NS
S3_CHUNK = 160
S4_PER_TILE = ETP // NS
S4_CHUNK = 160
ROWS_PER_TILE = N // NS
ZROWS = 125
IGNORED = N

_MESH = plsc.VectorSubcoreMesh(core_axis_name="c", subcore_axis_name="s")


def _s1_body(table, idx_hbm, out_hbm, iv, r0, r1, semg0, semg1, sw0, sw1):
    c = lax.axis_index("c")
    s = lax.axis_index("s")
    wid = s * NC + c
    tb = wid * S1_PER_W
    NCH = S1_PER_W // S1_CHUNK
    rbs = (r0, r1)
    semg = (semg0, semg1)
    semw = (sw0, sw1)
    pltpu.sync_copy(idx_hbm.at[pl.ds(tb, S1_PER_W)], iv)

    def issue(i, b):
        pltpu.async_copy(table.at[iv.at[pl.ds(i * S1_CHUNK, S1_CHUNK)]],
                         rbs[b], semg[b])

    issue(0, 0)
    issue(1, 1)

    def outer(i2, carry):
        for b in range(2):
            i = i2 * 2 + b
            pltpu.make_async_copy(table.at[pl.ds(0, S1_CHUNK), :], rbs[b],
                                  semg[b]).wait()
            pltpu.async_copy(rbs[b],
                             out_hbm.at[pl.ds(tb + i * S1_CHUNK, S1_CHUNK), :],
                             semw[b])

            @pl.when(i + 2 < NCH)
            def _():
                pltpu.make_async_copy(
                    rbs[b], out_hbm.at[pl.ds(0, S1_CHUNK), :], semw[b]).wait()
                issue(i + 2, b)

        return carry

    lax.fori_loop(0, NCH // 2, outer, 0)
    for b in range(2):
        pltpu.make_async_copy(
            rbs[b], out_hbm.at[pl.ds(0, S1_CHUNK), :], semw[b]).wait()


_s1 = pl.kernel(
    _s1_body,
    out_type=jax.ShapeDtypeStruct((NT_PAD, D), jnp.float32),
    mesh=_MESH,
    compiler_params=pltpu.CompilerParams(
        needs_layout_passes=False, use_tc_tiling_on_sc=False),
    scratch_types=[
        pltpu.VMEM((S1_PER_W,), jnp.int32),
        pltpu.VMEM((S1_CHUNK, D), jnp.float32),
        pltpu.VMEM((S1_CHUNK, D), jnp.float32),
        pltpu.SemaphoreType.DMA,
        pltpu.SemaphoreType.DMA,
        pltpu.SemaphoreType.DMA,
        pltpu.SemaphoreType.DMA,
    ],
)


def _s2_body(q_hbm, k_hbm, qidx_hbm, kidx_hbm, ex_hbm, qiv, kiv,
             qb0, qb1, kb0, kb1, exb0, exb1, mini, semg0, semg1, semx):
    c = lax.axis_index("c")
    s = lax.axis_index("s")
    wid = s * NC + c
    inv_sqrt_d = 1.0 / (float(D) ** 0.5)
    lanes = lax.iota(jnp.int32, L)
    zc = jnp.zeros((L,), jnp.int32)
    tbase = wid * S2_PER_W
    NCH = S2_PER_W // S2_CHUNK
    qbs = (qb0, qb1)
    kbs = (kb0, kb1)
    exbs = (exb0, exb1)
    semg = (semg0, semg1)

    pltpu.sync_copy(qidx_hbm.at[pl.ds(tbase, S2_PER_W)], qiv)
    pltpu.sync_copy(kidx_hbm.at[pl.ds(tbase, S2_PER_W)], kiv)

    def issue(i, b):
        pltpu.async_copy(q_hbm.at[qiv.at[pl.ds(i * S2_CHUNK, S2_CHUNK)]],
                         qbs[b], semg[b])
        pltpu.async_copy(k_hbm.at[kiv.at[pl.ds(i * S2_CHUNK, S2_CHUNK)]],
                         kbs[b], semg[b])

    issue(0, 0)
    issue(1, 1)

    def outer(i2, carry):
        for b in range(2):
            i = i2 * 2 + b
            base = tbase + i * S2_CHUNK

            @pl.when(i >= 2)
            def _():
                pltpu.make_async_copy(
                    exbs[b], ex_hbm.at[pl.ds(tbase, S2_CHUNK), :], semx).wait()

            pltpu.make_async_copy(q_hbm.at[pl.ds(0, S2_CHUNK), :], qbs[b],
                                  semg[b]).wait()
            pltpu.make_async_copy(k_hbm.at[pl.ds(0, S2_CHUNK), :], kbs[b],
                                  semg[b]).wait()

            def group(g, carry2):
                e0 = g * L
                for e in range(L):
                    rowv = zc + (e0 + e)
                    acc = jnp.zeros((L,), jnp.float32)
                    for j in range(D // L):
                        cols = lanes + j * L
                        qv = plsc.load_gather(qbs[b], [rowv, cols])
                        kv = plsc.load_gather(kbs[b], [rowv, cols])
                        acc = acc + qv * kv
                    plsc.store_scatter(
                        mini, [jnp.full((L,), e, jnp.int32), lanes], acc)
                tot = jnp.zeros((L,), jnp.float32)
                for ccol in range(L):
                    tot = tot + plsc.load_gather(
                        mini, [lanes, jnp.full((L,), ccol, jnp.int32)])
                exv = jnp.exp(tot * inv_sqrt_d)
                plsc.store_scatter(exbs[b], [lanes + e0, zc], exv)
                return carry2

            lax.fori_loop(0, S2_CHUNK // L, group, 0)
            pltpu.async_copy(exbs[b], ex_hbm.at[pl.ds(base, S2_CHUNK), :],
                             semx)

            @pl.when(i + 2 < NCH)
            def _():
                issue(i + 2, b)

        return carry

    lax.fori_loop(0, NCH // 2, outer, 0)
    for b in range(2):
        pltpu.make_async_copy(
            exbs[b], ex_hbm.at[pl.ds(tbase, S2_CHUNK), :], semx).wait()


_s2 = pl.kernel(
    _s2_body,
    out_type=jax.ShapeDtypeStruct((ETP, 1), jnp.float32),
    mesh=_MESH,
    compiler_params=pltpu.CompilerParams(
        needs_layout_passes=False, use_tc_tiling_on_sc=False),
    scratch_types=[
        pltpu.VMEM((S2_PER_W,), jnp.int32),
        pltpu.VMEM((S2_PER_W,), jnp.int32),
        pltpu.VMEM((S2_CHUNK, D), jnp.float32),
        pltpu.VMEM((S2_CHUNK, D), jnp.float32),
        pltpu.VMEM((S2_CHUNK, D), jnp.float32),
        pltpu.VMEM((S2_CHUNK, D), jnp.float32),
        pltpu.VMEM((S2_CHUNK, 1), jnp.float32),
        pltpu.VMEM((S2_CHUNK, 1), jnp.float32),
        pltpu.VMEM((L, 17), jnp.float32),
        pltpu.SemaphoreType.DMA,
        pltpu.SemaphoreType.DMA,
        pltpu.SemaphoreType.DMA,
    ],
)


def _s3_body(v2, esrc, edst2, ex2, zrow, zden, numer, den,
             siv, div, exv_pre, vb0, vb1, eb0, eb1,
             accum, dacc, semg0, semg1, sv0, sv1, se0, se1):
    c = lax.axis_index("c")
    s = lax.axis_index("s")
    row0 = s * ROWS_PER_TILE
    lanes = lax.iota(jnp.int32, L)
    zc = jnp.zeros((L,), jnp.int32)
    vbs = (vb0, vb1)
    ebs = (eb0, eb1)
    semg = (semg0, semg1)
    semv = (sv0, sv1)
    seme = (se0, se1)
    NCH = S3_PER_TILE // S3_CHUNK

    for b in range(2):
        pltpu.sync_copy(zden.at[pl.ds(0, S3_CHUNK), :], ebs[b])

    def issue_gather(j, b):
        pltpu.async_copy(
            v2.at[c].at[siv.at[pl.ds(j * S3_CHUNK, S3_CHUNK)]], vbs[b],
            semg[b])

    def per_type(t, carry):
        plsc.subcore_barrier()
        pltpu.sync_copy(zrow, accum.at[pl.ds(row0, ROWS_PER_TILE), :])

        @pl.when(c == 0)
        def _():
            pltpu.sync_copy(zden, dacc.at[pl.ds(row0, ROWS_PER_TILE), :])

        tb = t * EP + s * S3_PER_TILE
        pltpu.sync_copy(esrc.at[pl.ds(tb, S3_PER_TILE)], siv)
        pltpu.sync_copy(
            edst2.at[pl.ds(t * (EP // S3_CHUNK) + s * NCH, NCH), :], div)
        pltpu.sync_copy(ex2.at[pl.ds(tb, S3_PER_TILE)], exv_pre)
        plsc.subcore_barrier()
        issue_gather(0, 0)
        issue_gather(1, 1)

        def outer(j2, carry2):
            for b in range(2):
                j = j2 * 2 + b
                cb = j * S3_CHUNK
                pltpu.make_async_copy(
                    v2.at[c].at[pl.ds(0, S3_CHUNK)], vbs[b], semg[b]).wait()

                @pl.when(c == 0)
                def _():
                    def g_body(g, carry3):
                        rows = lanes + g * L
                        exv = plsc.load_gather(exv_pre, [rows + cb])
                        plsc.store_scatter(ebs[b], [rows, zc], exv)
                        return carry3

                    lax.fori_loop(0, S3_CHUNK // L, g_body, 0)

                def edge_body(e, carry3):
                    rowv = zc + e
                    exsplat = plsc.load_gather(exv_pre, [zc + (cb + e)])
                    for j4 in range(64 // L):
                        cols = lanes + j4 * L
                        vv = plsc.load_gather(vbs[b], [rowv, cols])
                        plsc.store_scatter(vbs[b], [rowv, cols], vv * exsplat)
                    return carry3

                lax.fori_loop(0, S3_CHUNK, edge_body, 0, unroll=2)
                pltpu.async_copy(
                    vbs[b],
                    accum.at[plsc.Indices(div.at[j], ignored_value=IGNORED)],
                    semv[b], add=True)

                @pl.when(c == 0)
                def _():
                    pltpu.async_copy(
                        ebs[b],
                        dacc.at[plsc.Indices(div.at[j],
                                             ignored_value=IGNORED)],
                        seme[b], add=True)

                @pl.when(j + 2 < NCH)
                def _():
                    pltpu.make_async_copy(
                        vbs[b], accum.at[pl.ds(0, S3_CHUNK), :],
                        semv[b]).wait()

                    @pl.when(c == 0)
                    def _():
                        pltpu.make_async_copy(
                            ebs[b], dacc.at[pl.ds(0, S3_CHUNK), :],
                            seme[b]).wait()

                    issue_gather(j + 2, b)

            return carry2

        lax.fori_loop(0, NCH // 2, outer, 0)
        for b in range(2):
            pltpu.make_async_copy(
                vbs[b], accum.at[pl.ds(0, S3_CHUNK), :], semv[b]).wait()

            @pl.when(c == 0)
            def _():
                pltpu.make_async_copy(
                    ebs[b], dacc.at[pl.ds(0, S3_CHUNK), :], seme[b]).wait()

        plsc.subcore_barrier()
        pltpu.sync_copy(accum.at[pl.ds(row0, ROWS_PER_TILE), :],
                        numer.at[c, t, pl.ds(row0, ROWS_PER_TILE), :])

        @pl.when(c == 0)
        def _():
            pltpu.sync_copy(dacc.at[pl.ds(row0, ROWS_PER_TILE), :],
                            den.at[t, pl.ds(row0, ROWS_PER_TILE), :])

        return carry

    lax.fori_loop(0, T, per_type, 0)


_s3 = pl.kernel(
    _s3_body,
    out_type=(
        jax.ShapeDtypeStruct((NC, T, N, 64), jnp.float32),
        jax.ShapeDtypeStruct((T, N, 8), jnp.float32),
    ),
    mesh=_MESH,
    compiler_params=pltpu.CompilerParams(
        needs_layout_passes=False, use_tc_tiling_on_sc=False),
    scratch_types=[
        pltpu.VMEM((S3_PER_TILE,), jnp.int32),
        pltpu.VMEM((S3_PER_TILE // S3_CHUNK, S3_CHUNK), jnp.int32),
        pltpu.VMEM((S3_PER_TILE,), jnp.float32),
        pltpu.VMEM((S3_CHUNK, 64), jnp.float32),
        pltpu.VMEM((S3_CHUNK, 64), jnp.float32),
        pltpu.VMEM((S3_CHUNK, 8), jnp.float32),
        pltpu.VMEM((S3_CHUNK, 8), jnp.float32),
        pltpu.VMEM_SHARED((N, 64), jnp.float32),
        pltpu.VMEM_SHARED((N, 8), jnp.float32),
        pltpu.SemaphoreType.DMA,
        pltpu.SemaphoreType.DMA,
        pltpu.SemaphoreType.DMA,
        pltpu.SemaphoreType.DMA,
        pltpu.SemaphoreType.DMA,
        pltpu.SemaphoreType.DMA,
    ],
)


_S4_SEC = 5
_S4_CPS = (S4_PER_TILE // S4_CHUNK) // _S4_SEC


def _s4_body(p2, esrc, edst2, zrow, agg, siv, div, pb0, pb1,
             accum, semg0, semg1, sv0, sv1):
    c = lax.axis_index("c")
    s = lax.axis_index("s")
    row0 = s * ROWS_PER_TILE
    pltpu.sync_copy(zrow, accum.at[pl.ds(row0, ROWS_PER_TILE), :])
    plsc.subcore_barrier()
    pbs = (pb0, pb1)
    semg = (semg0, semg1)
    semv = (sv0, sv1)
    epsec = _S4_CPS * S4_CHUNK

    def issue(j, b):
        pltpu.async_copy(
            p2.at[c].at[siv.at[pl.ds(j * S4_CHUNK, S4_CHUNK)]], pbs[b],
            semg[b])

    def section(sec, carry):
        sbase = s * S4_PER_TILE + sec * epsec
        pltpu.sync_copy(esrc.at[pl.ds(sbase, epsec)], siv)
        pltpu.sync_copy(
            edst2.at[pl.ds(s * (S4_PER_TILE // S4_CHUNK) + sec * _S4_CPS,
                           _S4_CPS), :], div)
        issue(0, 0)
        issue(1, 1)

        def outer(j2, carry2):
            for b in range(2):
                j = j2 * 2 + b
                pltpu.make_async_copy(
                    p2.at[c].at[pl.ds(0, S4_CHUNK)], pbs[b], semg[b]).wait()
                pltpu.async_copy(
                    pbs[b],
                    accum.at[plsc.Indices(div.at[j], ignored_value=IGNORED)],
                    semv[b], add=True)

                @pl.when(j + 2 < _S4_CPS)
                def _():
                    pltpu.make_async_copy(
                        pbs[b], accum.at[pl.ds(0, S4_CHUNK), :],
                        semv[b]).wait()
                    issue(j + 2, b)

            return carry2

        lax.fori_loop(0, _S4_CPS // 2, outer, 0)
        for b in range(2):
            pltpu.make_async_copy(
                pbs[b], accum.at[pl.ds(0, S4_CHUNK), :], semv[b]).wait()
        return carry

    lax.fori_loop(0, _S4_SEC, section, 0)
    plsc.subcore_barrier()
    pltpu.sync_copy(accum.at[pl.ds(row0, ROWS_PER_TILE), :],
                    agg.at[c, pl.ds(row0, ROWS_PER_TILE), :])


_s4 = pl.kernel(
    _s4_body,
    out_type=jax.ShapeDtypeStruct((NC, N, 64), jnp.float32),
    mesh=_MESH,
    compiler_params=pltpu.CompilerParams(
        needs_layout_passes=False, use_tc_tiling_on_sc=False),
    scratch_types=[
        pltpu.VMEM((_S4_CPS * S4_CHUNK,), jnp.int32),
        pltpu.VMEM((_S4_CPS, S4_CHUNK), jnp.int32),
        pltpu.VMEM((S4_CHUNK, 64), jnp.float32),
        pltpu.VMEM((S4_CHUNK, 64), jnp.float32),
        pltpu.VMEM_SHARED((N, 64), jnp.float32),
        pltpu.SemaphoreType.DMA,
        pltpu.SemaphoreType.DMA,
        pltpu.SemaphoreType.DMA,
        pltpu.SemaphoreType.DMA,
    ],
)


_T1B = 2048


def _t1_body(x_ref, w_ref, b_ref, q_ref, k_ref, v2_ref, s_ref):
    y = jnp.dot(x_ref[...], w_ref[...],
                preferred_element_type=jnp.float32) + b_ref[...]
    q_ref[...] = y[:, :128]
    k_ref[...] = y[:, 128:256]
    v2_ref[0] = y[:, 256:320]
    v2_ref[1] = y[:, 320:384]
    s_ref[...] = y[:, 384:]


def _t1(h0, wt, bcat):
    grid = (NT_PAD // _T1B,)
    return pl.pallas_call(
        _t1_body,
        grid=grid,
        in_specs=[
            pl.BlockSpec((_T1B, D), lambda i: (i, 0)),
            pl.BlockSpec((D, 512), lambda i: (0, 0)),
            pl.BlockSpec((1, 512), lambda i: (0, 0)),
        ],
        out_specs=[
            pl.BlockSpec((_T1B, D), lambda i: (i, 0)),
            pl.BlockSpec((_T1B, D), lambda i: (i, 0)),
            pl.BlockSpec((NC, _T1B, 64), lambda i: (0, i, 0)),
            pl.BlockSpec((_T1B, D), lambda i: (i, 0)),
        ],
        out_shape=[
            jax.ShapeDtypeStruct((NT_PAD, D), jnp.float32),
            jax.ShapeDtypeStruct((NT_PAD, D), jnp.float32),
            jax.ShapeDtypeStruct((NC, NT_PAD, 64), jnp.float32),
            jax.ShapeDtypeStruct((NT_PAD, D), jnp.float32),
        ],
    )(h0, wt, bcat)


_T2B = 2000


def _t2_body(n_ref, d_ref, s_ref, g_ref, b_ref, w_ref, p_ref):
    nl = n_ref[0, 0]
    nh = n_ref[1, 0]
    den = d_ref[0, :, :1] + 1e-16
    h = jnp.concatenate([nl, nh], axis=-1) / den + s_ref[0]
    mu = jnp.mean(h, axis=-1, keepdims=True)
    r = h - mu
    var = jnp.mean(r * r, axis=-1, keepdims=True)
    hn = r / jnp.sqrt(var + 1e-5) * g_ref[0, 0] + b_ref[0, 0]
    p = jnp.dot(hn, w_ref[0], preferred_element_type=jnp.float32)
    p_ref[0, 0] = p[:, :64]
    p_ref[1, 0] = p[:, 64:]


def _t2(numer, den, sres, g, b, w1t):
    grid = (T, N // _T2B)
    return pl.pallas_call(
        _t2_body,
        grid=grid,
        in_specs=[
            pl.BlockSpec((NC, 1, _T2B, 64), lambda t, r: (0, t, r, 0)),
            pl.BlockSpec((1, _T2B, 8), lambda t, r: (t, r, 0)),
            pl.BlockSpec((1, _T2B, D), lambda t, r: (t, r, 0)),
            pl.BlockSpec((1, 1, D), lambda t, r: (t, 0, 0)),
            pl.BlockSpec((1, 1, D), lambda t, r: (t, 0, 0)),
            pl.BlockSpec((1, D, D), lambda t, r: (t, 0, 0)),
        ],
        out_specs=pl.BlockSpec((NC, 1, _T2B, 64), lambda t, r: (0, t, r, 0)),
        out_shape=jax.ShapeDtypeStruct((NC, T, N, 64), jnp.float32),
    )(numer, den, sres, g, b, w1t)


_T3B = 2000


def _t3_body(x_ref, w1_ref, b1_ref, a_ref, w2_ref, b2_ref, w3_ref, b3_ref,
             o_ref):
    i = pl.program_id(0)
    acc = jnp.dot(x_ref[...], w1_ref[...],
                  preferred_element_type=jnp.float32) + b1_ref[...]
    acc = acc + jnp.where(i < N // _T3B, 1.0, 0.0) * a_ref[...]
    z = jnp.maximum(acc, 0.0)
    z2 = jnp.maximum(
        jnp.dot(z, w2_ref[...], preferred_element_type=jnp.float32)
        + b2_ref[...], 0.0)
    o_ref[...] = jnp.sum(z2 * w3_ref[...], axis=-1, keepdims=True) + b3_ref[0, 0]


def _t3(tx_x, w1a, b1, aggf, w2t, b2, w3, b3):
    grid = (NTX // _T3B,)
    return pl.pallas_call(
        _t3_body,
        grid=grid,
        in_specs=[
            pl.BlockSpec((_T3B, F), lambda i: (i, 0)),
            pl.BlockSpec((F, D), lambda i: (0, 0)),
            pl.BlockSpec((1, D), lambda i: (0, 0)),
            pl.BlockSpec((_T3B, D), lambda i: (jnp.minimum(i, N // _T3B - 1), 0)),
            pl.BlockSpec((D, 64), lambda i: (0, 0)),
            pl.BlockSpec((1, 64), lambda i: (0, 0)),
            pl.BlockSpec((1, 64), lambda i: (0, 0)),
            pl.BlockSpec((1, 1), lambda i: (0, 0)),
        ],
        out_specs=pl.BlockSpec((_T3B, 1), lambda i: (i, 0)),
        out_shape=jax.ShapeDtypeStruct((NTX, 1), jnp.float32),
    )(tx_x, w1a, b1, aggf, w2t, b2, w3, b3)


def kernel(tx_x, entity_x, edge_index, emb_tables, tc_Wq, tc_bq, tc_Wk, tc_bk,
           tc_Wv, tc_bv, tc_Ws, tc_bs, ln_g, ln_b, cls_W1, cls_b1, cls_W2,
           cls_b2, cls_W3, cls_b3):
    f32 = jnp.float32
    offs = (jnp.arange(T, dtype=jnp.int32) * N)[:, None]

    flat_idx = (entity_x + offs).reshape(-1)
    flat_idx = jnp.pad(flat_idx, (0, NT_PAD - NT))
    table = emb_tables.reshape(NT, D)

    gsrc = jnp.pad(edge_index[:, 0, :] + offs, ((0, 0), (0, EP - E))).reshape(-1)
    gdst = jnp.pad(edge_index[:, 1, :] + offs, ((0, 0), (0, EP - E))).reshape(-1)
    edst = jnp.pad(edge_index[:, 1, :], ((0, 0), (0, EP - E)),
                   constant_values=IGNORED).reshape(-1)

    h0 = _s1(table, flat_idx)

    wt = jnp.concatenate([tc_Wq, tc_Wk, tc_Wv, tc_Ws], axis=0).T
    bcat = jnp.concatenate([tc_bq, tc_bk, tc_bv, tc_bs])[None]
    q, k, v2, sres = _t1(h0, wt, bcat)

    ex2 = _s2(q, k, gdst, gsrc)

    zrow = jnp.zeros((ROWS_PER_TILE, 64), f32)
    zden = jnp.zeros((ROWS_PER_TILE, 8), f32)
    numer, den = _s3(v2, gsrc, edst.reshape(-1, S3_CHUNK), ex2.reshape(-1),
                     zrow, zden)

    sres_t = sres[:NT].reshape(T, N, D)
    w1t = cls_W1[:, F:].reshape(D, T, D).transpose(1, 2, 0)
    p2 = _t2(numer, den, sres_t, ln_g[:, None], ln_b[:, None], w1t)

    p2f = p2.reshape(NC, NT, 64)
    agg = _s4(p2f, gsrc, edst.reshape(-1, S4_CHUNK), zrow)
    aggf = jnp.concatenate([agg[0], agg[1]], axis=-1)

    return _t3(tx_x, cls_W1[:, :F].T, cls_b1[None], aggf, cls_W2.T,
               cls_b2[None], cls_W3, cls_b3[None])
